# baseline (speedup 1.0000x reference)
import functools

import jax
import jax.numpy as jnp
from jax import lax
from jax.experimental import pallas as pl
from jax.experimental.pallas import tpu as pltpu
from jax.experimental.pallas import tpu_sc as plsc

NODE_NUM = 1000000
HIDDEN_DIM = 64
BATCH = 16384

_info = plsc.get_sparse_core_info()
_NC, _NS = _info.num_cores, _info.num_subcores
_NW = _NC * _NS
_B_PER_W = BATCH // _NW
_HALF = _B_PER_W // 2
_CHUNK = 16


@functools.partial(
    pl.kernel,
    mesh=plsc.VectorSubcoreMesh(core_axis_name="c", subcore_axis_name="s"),
    out_type=jax.ShapeDtypeStruct((BATCH, HIDDEN_DIM), jnp.float32),
    scratch_types=[
        pltpu.VMEM((_B_PER_W,), jnp.int32),
        pltpu.VMEM((_HALF, HIDDEN_DIM), jnp.float32),
        pltpu.VMEM_SHARED((_NS * _HALF, HIDDEN_DIM), jnp.float32),
        pltpu.SemaphoreType.DMA,
        pltpu.SemaphoreType.DMA,
    ],
)
def _gather_kernel(idx_hbm, table_hbm, out_hbm, idx_v, rows_v, shared_v, sem_a, sem_b):
    wid = lax.axis_index("s") * _NC + lax.axis_index("c")
    sid = lax.axis_index("s")
    base = wid * _B_PER_W
    sbase = sid * _HALF
    pltpu.sync_copy(idx_hbm.at[pl.ds(base, _B_PER_W)], idx_v)

    @pl.loop(0, _HALF // _CHUNK)
    def _fire(i):
        va = idx_v[pl.ds(i * _CHUNK, _CHUNK)]
        vb = idx_v[pl.ds(_HALF + i * _CHUNK, _CHUNK)]
        for t in range(_CHUNK):
            j = i * _CHUNK + t
            pltpu.make_async_copy(
                table_hbm.at[pl.ds(va[t], 1), :],
                rows_v.at[pl.ds(j, 1), :],
                sem_a,
            ).start()
            pltpu.make_async_copy(
                table_hbm.at[pl.ds(vb[t], 1), :],
                shared_v.at[pl.ds(sbase + j, 1), :],
                sem_b,
            ).start()

    pltpu.make_async_copy(
        table_hbm.at[pl.ds(0, _HALF), :], rows_v, sem_a
    ).wait()
    pltpu.make_async_copy(
        table_hbm.at[pl.ds(0, _HALF), :],
        shared_v.at[pl.ds(sbase, _HALF), :],
        sem_b,
    ).wait()
    pltpu.sync_copy(rows_v, out_hbm.at[pl.ds(base, _HALF)])
    pltpu.sync_copy(
        shared_v.at[pl.ds(sbase, _HALF), :],
        out_hbm.at[pl.ds(base + _HALF, _HALF)],
    )


def kernel(indices, table):
    return _gather_kernel(indices.astype(jnp.int32), table)
